# Initial kernel scaffold; baseline (speedup 1.0000x reference)
import jax
import jax.numpy as jnp
from jax.experimental import pallas as pl


def kernel(x, difficulty_labels, W_experts, b_experts, emb, gate_W, gate_b):
    raise NotImplementedError("write your pallas kernel here")


# speedup vs baseline: 2.7575x; 2.7575x over previous
import functools

import jax
import jax.numpy as jnp
from jax.experimental import pallas as pl
from jax.experimental.pallas import tpu as pltpu

B = 8192
D = 1024
H = 1024
E = 8
ND = 3
TOPK = 2

BT = 256


def _fused_body(x_ref, lab_ref, w_ref, b_ref, emb_ref, gw_ref, gb_ref,
                out_ref, topi_ref):
    x = x_ref[...]
    gw = gw_ref[...]
    gwx = gw[:, :D].astype(jnp.bfloat16)
    gwd = gw[:, D:].astype(jnp.bfloat16)
    de_log = jax.lax.dot_general(emb_ref[...].astype(jnp.bfloat16), gwd,
                                 (((1,), (1,)), ((), ())),
                                 preferred_element_type=jnp.float32)
    lab = lab_ref[...]
    de = jnp.where(lab == 0, de_log[0][None, :],
                   jnp.where(lab == 1, de_log[1][None, :],
                             de_log[2][None, :]))
    logits = (jax.lax.dot_general(x.astype(jnp.bfloat16), gwx,
                                  (((1,), (1,)), ((), ())),
                                  preferred_element_type=jnp.float32)
              + de + gb_ref[...])
    eiota = jax.lax.broadcasted_iota(jnp.int32, (BT, E), 1)
    v1 = jnp.max(logits, axis=1, keepdims=True)
    i1 = jnp.min(jnp.where(logits == v1, eiota, E), axis=1,
                 keepdims=True)
    masked = jnp.where(eiota == i1, -jnp.inf, logits)
    v2 = jnp.max(masked, axis=1, keepdims=True)
    i2 = jnp.min(jnp.where(masked == v2, eiota, E), axis=1, keepdims=True)
    e21 = jnp.exp(v2 - v1)
    p1 = 1.0 / (1.0 + e21)
    p2 = e21 / (1.0 + e21)
    topi_ref[...] = jnp.concatenate([i1, i2], axis=1)

    acc = jnp.zeros((BT, H), jnp.float32)
    for e in range(E):
        we = p1 * (i1 == e) + p2 * (i2 == e)
        mm = jax.lax.dot_general(x, w_ref[e], (((1,), (1,)), ((), ())))
        acc = acc + we * (mm + b_ref[e][None, :])
    out_ref[...] = acc


@jax.jit
def _fused(x, labels, W_experts, b_experts, emb, gate_W, gate_b):
    grid = (B // BT,)
    return pl.pallas_call(
        _fused_body,
        grid=grid,
        in_specs=[
            pl.BlockSpec((BT, D), lambda t: (t, 0)),
            pl.BlockSpec((BT, 1), lambda t: (t, 0)),
            pl.BlockSpec((E, H, D), lambda t: (0, 0, 0)),
            pl.BlockSpec((E, H), lambda t: (0, 0)),
            pl.BlockSpec((ND, D), lambda t: (0, 0)),
            pl.BlockSpec((E, 2 * D), lambda t: (0, 0)),
            pl.BlockSpec((1, E), lambda t: (0, 0)),
        ],
        out_specs=[
            pl.BlockSpec((BT, H), lambda t: (t, 0)),
            pl.BlockSpec((BT, TOPK), lambda t: (t, 0)),
        ],
        out_shape=[
            jax.ShapeDtypeStruct((B, H), jnp.float32),
            jax.ShapeDtypeStruct((B, TOPK), jnp.int32),
        ],
    )(x, labels, W_experts, b_experts, emb, gate_W, gate_b)


def kernel(x, difficulty_labels, W_experts, b_experts, emb, gate_W, gate_b):
    lab = difficulty_labels.astype(jnp.int32).reshape(B, 1)
    gb = gate_b.reshape(1, E)
    out, topi = _fused(x, lab, W_experts, b_experts, emb, gate_W, gb)
    return (out, topi)
